# baseline (speedup 1.0000x reference)
import functools

import jax
import jax.numpy as jnp
from jax import lax
from jax.experimental import pallas as pl
from jax.experimental.pallas import tpu as pltpu
from jax.experimental.pallas import tpu_sc as plsc

N_BINS = 15
_NW = 32
_LANES = 16
_NPIECE = 4


def _piece_sizes(total, nb):
    per = total // nb // _LANES * _LANES
    sizes = [per] * nb
    rem = total - per * nb
    i = 0
    while rem > 0:
        sizes[i % nb] += _LANES
        rem -= _LANES
        i += 1
    return [s for s in sizes if s > 0]


def _sc_hist(conf, pred, lab, ch, tail):
    sizes = _piece_sizes(ch, _NPIECE)
    offs = [sum(sizes[:i]) for i in range(len(sizes))]
    ps = max(sizes)
    ntail = tail // _LANES
    mesh = plsc.VectorSubcoreMesh(core_axis_name="c", subcore_axis_name="s")
    hsz = 2 * N_BINS * _LANES

    @functools.partial(
        pl.kernel, mesh=mesh,
        out_type=jax.ShapeDtypeStruct((_NW * hsz,), jnp.float32),
        scratch_types=[
            pltpu.VMEM((ps,), jnp.float32),
            pltpu.VMEM((ps,), jnp.int32),
            pltpu.VMEM((ps,), jnp.int32),
            pltpu.VMEM((ps,), jnp.float32),
            pltpu.VMEM((ps,), jnp.int32),
            pltpu.VMEM((ps,), jnp.int32),
            pltpu.VMEM((max(tail, _LANES),), jnp.float32),
            pltpu.VMEM((max(tail, _LANES),), jnp.int32),
            pltpu.VMEM((max(tail, _LANES),), jnp.int32),
            pltpu.VMEM((hsz,), jnp.float32),
            pltpu.SemaphoreType.DMA,
            pltpu.SemaphoreType.DMA,
        ],
        compiler_params=pltpu.CompilerParams(needs_layout_passes=False),
    )
    def k(conf_hbm, pred_hbm, lab_hbm, out_hbm,
          conf_a, pred_a, lab_a, conf_b, pred_b, lab_b,
          tconf_v, tpred_v, tlab_v, hist_v, sem0, sem1):
        sems = [sem0, sem1]
        bufs = [(conf_a, pred_a, lab_a), (conf_b, pred_b, lab_b)]
        wid = lax.axis_index("s") * 2 + lax.axis_index("c")
        base = wid * ch

        def start_piece(p, slot):
            off = base + offs[p]
            sz = sizes[p]
            cv, pv, lv = bufs[slot]
            return [
                pltpu.async_copy(conf_hbm.at[pl.ds(off, sz)],
                                 cv.at[pl.ds(0, sz)], sems[slot]),
                pltpu.async_copy(pred_hbm.at[pl.ds(off, sz)],
                                 pv.at[pl.ds(0, sz)], sems[slot]),
                pltpu.async_copy(lab_hbm.at[pl.ds(off, sz)],
                                 lv.at[pl.ds(0, sz)], sems[slot]),
            ]

        handles = start_piece(0, 0)

        zf = jnp.zeros((_LANES,), jnp.float32)
        for r in range(2 * N_BINS):
            hist_v[pl.ds(r * _LANES, _LANES)] = zf
        col = lax.iota(jnp.int32, _LANES)
        onesf = jnp.ones((_LANES,), jnp.float32)
        onespk = jnp.full((_LANES,), 1.0 + 1.0 / 4096.0, jnp.float32)
        i0 = jnp.zeros((_LANES,), jnp.int32)
        i14 = jnp.full((_LANES,), N_BINS - 1, jnp.int32)
        i240 = jnp.full((_LANES,), N_BINS * _LANES, jnp.int32)
        f15 = jnp.full((_LANES,), 15.0, jnp.float32)

        def chunk(cref, pref, lref, s):
            c = cref[pl.ds(s, _LANES)]
            p = pref[pl.ds(s, _LANES)]
            l = lref[pl.ds(s, _LANES)]
            v = jnp.where(p == l, onespk, onesf)
            t = (c * f15).astype(jnp.int32)
            valid = c > zf
            row = jnp.minimum(jnp.maximum(t, i0), i14)
            flat = row * _LANES + col
            plsc.addupdate_scatter(hist_v, [flat], v, mask=valid)
            plsc.addupdate_scatter(hist_v, [flat + i240], c, mask=valid)

        for p in range(len(sizes)):
            slot = p % 2
            for h in handles:
                h.wait()
            if p + 1 < len(sizes):
                handles = start_piece(p + 1, (p + 1) % 2)
            cref, pref, lref = bufs[slot]

            @plsc.parallel_loop(0, sizes[p], step=_LANES, unroll=16)
            def _(s):
                chunk(cref, pref, lref, s)

        if ntail:
            @pl.when(wid == _NW - 1)
            def _():
                tbase = _NW * ch
                pltpu.sync_copy(conf_hbm.at[pl.ds(tbase, tail)],
                                tconf_v.at[pl.ds(0, tail)])
                pltpu.sync_copy(pred_hbm.at[pl.ds(tbase, tail)],
                                tpred_v.at[pl.ds(0, tail)])
                pltpu.sync_copy(lab_hbm.at[pl.ds(tbase, tail)],
                                tlab_v.at[pl.ds(0, tail)])

                @plsc.parallel_loop(0, ntail * _LANES, step=_LANES)
                def _(s):
                    chunk(tconf_v, tpred_v, tlab_v, s)

        pltpu.sync_copy(hist_v, out_hbm.at[pl.ds(wid * hsz, hsz)])

    return k(conf, pred, lab)


def _ece_body(p_ref, o_ref, *, n_total):
    x = p_ref[...]
    pk = x[:, 0:N_BINS, :]
    cf = x[:, N_BINS:2 * N_BINS, :]
    cnt_e = jnp.floor(pk)
    acc_e = (pk - cnt_e) * 4096.0
    cnt = jnp.sum(jnp.sum(cnt_e, axis=0), axis=1, keepdims=True)
    sacc = jnp.sum(jnp.sum(acc_e, axis=0), axis=1, keepdims=True)
    sconf = jnp.sum(jnp.sum(cf, axis=0), axis=1, keepdims=True)
    safe = jnp.maximum(cnt, 1.0)
    prop = cnt / n_total
    contrib = jnp.abs(sconf / safe - sacc / safe) * prop
    ece = jnp.sum(jnp.where(prop > 0.0, contrib, 0.0))
    o_ref[...] = jnp.full((8, 128), ece, jnp.float32)


def kernel(confidences, predictions, labels):
    n = confidences.shape[0]
    conf = confidences.astype(jnp.float32)
    pred = predictions.astype(jnp.int32)
    lab = labels.astype(jnp.int32)
    if n % _LANES:
        npad = -(-n // _LANES) * _LANES
        conf = jnp.pad(conf, (0, npad - n))
        pred = jnp.pad(pred, (0, npad - n))
        lab = jnp.pad(lab, (0, npad - n))
        m = npad
    else:
        m = n
    ch = (m // (_NW * _LANES)) * _LANES
    tail = m - _NW * ch

    flat = _sc_hist(conf, pred, lab, ch, tail)
    parts = flat.reshape(_NW, 2 * N_BINS, _LANES)

    ece = pl.pallas_call(
        functools.partial(_ece_body, n_total=float(n)),
        out_shape=jax.ShapeDtypeStruct((8, 128), jnp.float32),
    )(parts)
    return ece[0, 0:1]


# speedup vs baseline: 1.1761x; 1.1761x over previous
import functools

import jax
import jax.numpy as jnp
from jax import lax
from jax.experimental import pallas as pl
from jax.experimental.pallas import tpu as pltpu
from jax.experimental.pallas import tpu_sc as plsc

N_BINS = 15
_NW = 32
_LANES = 16
_NPIECE = 4


def _piece_sizes(total, nb):
    per = total // nb // _LANES * _LANES
    sizes = [per] * nb
    rem = total - per * nb
    i = 0
    while rem > 0:
        sizes[i % nb] += _LANES
        rem -= _LANES
        i += 1
    return [s for s in sizes if s > 0]


def _sc_hist(conf, pred, lab, ch, tail):
    sizes = _piece_sizes(ch, _NPIECE)
    offs = [sum(sizes[:i]) for i in range(len(sizes))]
    ps = max(sizes)
    ntail = tail // _LANES
    mesh = plsc.VectorSubcoreMesh(core_axis_name="c", subcore_axis_name="s")
    hsz = 2 * N_BINS * _LANES

    @functools.partial(
        pl.kernel, mesh=mesh,
        out_type=jax.ShapeDtypeStruct((_NW * hsz,), jnp.float32),
        scratch_types=[
            pltpu.VMEM((ps,), jnp.float32),
            pltpu.VMEM((ps,), jnp.int32),
            pltpu.VMEM((ps,), jnp.int32),
            pltpu.VMEM((ps,), jnp.float32),
            pltpu.VMEM((ps,), jnp.int32),
            pltpu.VMEM((ps,), jnp.int32),
            pltpu.VMEM((max(tail, _LANES),), jnp.float32),
            pltpu.VMEM((max(tail, _LANES),), jnp.int32),
            pltpu.VMEM((max(tail, _LANES),), jnp.int32),
            pltpu.VMEM((hsz,), jnp.float32),
            pltpu.SemaphoreType.DMA,
            pltpu.SemaphoreType.DMA,
        ],
        compiler_params=pltpu.CompilerParams(needs_layout_passes=False),
    )
    def k(conf_hbm, pred_hbm, lab_hbm, out_hbm,
          conf_a, pred_a, lab_a, conf_b, pred_b, lab_b,
          tconf_v, tpred_v, tlab_v, hist_v, sem0, sem1):
        sems = [sem0, sem1]
        bufs = [(conf_a, pred_a, lab_a), (conf_b, pred_b, lab_b)]
        wid = lax.axis_index("s") * 2 + lax.axis_index("c")
        base = wid * ch

        def start_piece(p, slot):
            off = base + offs[p]
            sz = sizes[p]
            cv, pv, lv = bufs[slot]
            return [
                pltpu.async_copy(conf_hbm.at[pl.ds(off, sz)],
                                 cv.at[pl.ds(0, sz)], sems[slot]),
                pltpu.async_copy(pred_hbm.at[pl.ds(off, sz)],
                                 pv.at[pl.ds(0, sz)], sems[slot]),
                pltpu.async_copy(lab_hbm.at[pl.ds(off, sz)],
                                 lv.at[pl.ds(0, sz)], sems[slot]),
            ]

        handles = start_piece(0, 0)

        zf = jnp.zeros((_LANES,), jnp.float32)
        for r in range(2 * N_BINS):
            hist_v[pl.ds(r * _LANES, _LANES)] = zf
        col = lax.iota(jnp.int32, _LANES)
        onesf = jnp.ones((_LANES,), jnp.float32)
        onespk = jnp.full((_LANES,), 1.0 + 1.0 / 4096.0, jnp.float32)
        i0 = jnp.zeros((_LANES,), jnp.int32)
        i14 = jnp.full((_LANES,), N_BINS - 1, jnp.int32)
        i240 = jnp.full((_LANES,), N_BINS * _LANES, jnp.int32)
        f15 = jnp.full((_LANES,), 15.0, jnp.float32)

        def chunk(cref, pref, lref, s):
            c = cref[pl.ds(s, _LANES)]
            p = pref[pl.ds(s, _LANES)]
            l = lref[pl.ds(s, _LANES)]
            v = jnp.where(p == l, onespk, onesf)
            t = (c * f15).astype(jnp.int32)
            valid = c > zf
            row = jnp.minimum(jnp.maximum(t, i0), i14)
            flat = row * _LANES + col
            plsc.addupdate_scatter(hist_v, [flat], v, mask=valid)
            plsc.addupdate_scatter(hist_v, [flat + i240], c, mask=valid)

        for p in range(len(sizes)):
            slot = p % 2
            for h in handles:
                h.wait()
            if p + 1 < len(sizes):
                handles = start_piece(p + 1, (p + 1) % 2)
            cref, pref, lref = bufs[slot]

            @plsc.parallel_loop(0, sizes[p], step=_LANES, unroll=8)
            def _(s):
                chunk(cref, pref, lref, s)

        if ntail:
            @pl.when(wid == _NW - 1)
            def _():
                tbase = _NW * ch
                pltpu.sync_copy(conf_hbm.at[pl.ds(tbase, tail)],
                                tconf_v.at[pl.ds(0, tail)])
                pltpu.sync_copy(pred_hbm.at[pl.ds(tbase, tail)],
                                tpred_v.at[pl.ds(0, tail)])
                pltpu.sync_copy(lab_hbm.at[pl.ds(tbase, tail)],
                                tlab_v.at[pl.ds(0, tail)])

                @plsc.parallel_loop(0, ntail * _LANES, step=_LANES)
                def _(s):
                    chunk(tconf_v, tpred_v, tlab_v, s)

        pltpu.sync_copy(hist_v, out_hbm.at[pl.ds(wid * hsz, hsz)])

    return k(conf, pred, lab)


def _ece_body(p_ref, o_ref, *, n_total):
    x = p_ref[...]
    pk = x[:, 0:N_BINS, :]
    cf = x[:, N_BINS:2 * N_BINS, :]
    cnt_e = jnp.floor(pk)
    acc_e = (pk - cnt_e) * 4096.0
    cnt = jnp.sum(jnp.sum(cnt_e, axis=0), axis=1, keepdims=True)
    sacc = jnp.sum(jnp.sum(acc_e, axis=0), axis=1, keepdims=True)
    sconf = jnp.sum(jnp.sum(cf, axis=0), axis=1, keepdims=True)
    safe = jnp.maximum(cnt, 1.0)
    prop = cnt / n_total
    contrib = jnp.abs(sconf / safe - sacc / safe) * prop
    ece = jnp.sum(jnp.where(prop > 0.0, contrib, 0.0))
    o_ref[...] = jnp.full((8, 128), ece, jnp.float32)


def kernel(confidences, predictions, labels):
    n = confidences.shape[0]
    conf = confidences.astype(jnp.float32)
    pred = predictions.astype(jnp.int32)
    lab = labels.astype(jnp.int32)
    if n % _LANES:
        npad = -(-n // _LANES) * _LANES
        conf = jnp.pad(conf, (0, npad - n))
        pred = jnp.pad(pred, (0, npad - n))
        lab = jnp.pad(lab, (0, npad - n))
        m = npad
    else:
        m = n
    ch = (m // (_NW * _LANES)) * _LANES
    tail = m - _NW * ch

    flat = _sc_hist(conf, pred, lab, ch, tail)
    parts = flat.reshape(_NW, 2 * N_BINS, _LANES)

    ece = pl.pallas_call(
        functools.partial(_ece_body, n_total=float(n)),
        out_shape=jax.ShapeDtypeStruct((8, 128), jnp.float32),
    )(parts)
    return ece[0, 0:1]


# speedup vs baseline: 1.3089x; 1.1129x over previous
import functools

import jax
import jax.numpy as jnp
from jax import lax
from jax.experimental import pallas as pl
from jax.experimental.pallas import tpu as pltpu
from jax.experimental.pallas import tpu_sc as plsc

N_BINS = 15
_NW = 32
_LANES = 16
_NPIECE = 4


def _piece_sizes(total, nb):
    per = total // nb // _LANES * _LANES
    sizes = [per] * nb
    rem = total - per * nb
    i = 0
    while rem > 0:
        sizes[i % nb] += _LANES
        rem -= _LANES
        i += 1
    return [s for s in sizes if s > 0]


def _sc_hist(conf, pred, lab, ch, tail):
    sizes = _piece_sizes(ch, _NPIECE)
    offs = [sum(sizes[:i]) for i in range(len(sizes))]
    ps = max(sizes)
    ntail = tail // _LANES
    mesh = plsc.VectorSubcoreMesh(core_axis_name="c", subcore_axis_name="s")
    hsz = 512

    @functools.partial(
        pl.kernel, mesh=mesh,
        out_type=jax.ShapeDtypeStruct((_NW * hsz,), jnp.float32),
        scratch_types=[
            pltpu.VMEM((ps,), jnp.float32),
            pltpu.VMEM((ps,), jnp.int32),
            pltpu.VMEM((ps,), jnp.int32),
            pltpu.VMEM((ps,), jnp.float32),
            pltpu.VMEM((ps,), jnp.int32),
            pltpu.VMEM((ps,), jnp.int32),
            pltpu.VMEM((max(tail, _LANES),), jnp.float32),
            pltpu.VMEM((max(tail, _LANES),), jnp.int32),
            pltpu.VMEM((max(tail, _LANES),), jnp.int32),
            pltpu.VMEM((hsz,), jnp.float32),
            pltpu.SemaphoreType.DMA,
            pltpu.SemaphoreType.DMA,
        ],
        compiler_params=pltpu.CompilerParams(needs_layout_passes=False),
    )
    def k(conf_hbm, pred_hbm, lab_hbm, out_hbm,
          conf_a, pred_a, lab_a, conf_b, pred_b, lab_b,
          tconf_v, tpred_v, tlab_v, hist_v, sem0, sem1):
        sems = [sem0, sem1]
        bufs = [(conf_a, pred_a, lab_a), (conf_b, pred_b, lab_b)]
        wid = lax.axis_index("s") * 2 + lax.axis_index("c")
        base = wid * ch

        def start_piece(p, slot):
            off = base + offs[p]
            sz = sizes[p]
            cv, pv, lv = bufs[slot]
            return [
                pltpu.async_copy(conf_hbm.at[pl.ds(off, sz)],
                                 cv.at[pl.ds(0, sz)], sems[slot]),
                pltpu.async_copy(pred_hbm.at[pl.ds(off, sz)],
                                 pv.at[pl.ds(0, sz)], sems[slot]),
                pltpu.async_copy(lab_hbm.at[pl.ds(off, sz)],
                                 lv.at[pl.ds(0, sz)], sems[slot]),
            ]

        handles = start_piece(0, 0)

        zf = jnp.zeros((_LANES,), jnp.float32)
        for r in range(hsz // _LANES):
            hist_v[pl.ds(r * _LANES, _LANES)] = zf
        col = lax.iota(jnp.int32, _LANES)
        onesf = jnp.ones((_LANES,), jnp.float32)
        onespk = jnp.full((_LANES,), 1.0 + 1.0 / 4096.0, jnp.float32)
        i0 = jnp.zeros((_LANES,), jnp.int32)
        i14 = jnp.full((_LANES,), N_BINS - 1, jnp.int32)
        i240 = jnp.full((_LANES,), N_BINS * _LANES, jnp.int32)
        f15 = jnp.full((_LANES,), 15.0, jnp.float32)

        def chunk(cref, pref, lref, s):
            c = cref[pl.ds(s, _LANES)]
            p = pref[pl.ds(s, _LANES)]
            l = lref[pl.ds(s, _LANES)]
            v = jnp.where(p == l, onespk, onesf)
            t = (c * f15).astype(jnp.int32)
            valid = c > zf
            row = jnp.minimum(jnp.maximum(t, i0), i14)
            flat = row * _LANES + col
            plsc.addupdate_scatter(hist_v, [flat], v, mask=valid)
            plsc.addupdate_scatter(hist_v, [flat + i240], c, mask=valid)

        for p in range(len(sizes)):
            slot = p % 2
            for h in handles:
                h.wait()
            if p + 1 < len(sizes):
                handles = start_piece(p + 1, (p + 1) % 2)
            cref, pref, lref = bufs[slot]

            @plsc.parallel_loop(0, sizes[p], step=_LANES, unroll=8)
            def _(s):
                chunk(cref, pref, lref, s)

        if ntail:
            @pl.when(wid == _NW - 1)
            def _():
                tbase = _NW * ch
                pltpu.sync_copy(conf_hbm.at[pl.ds(tbase, tail)],
                                tconf_v.at[pl.ds(0, tail)])
                pltpu.sync_copy(pred_hbm.at[pl.ds(tbase, tail)],
                                tpred_v.at[pl.ds(0, tail)])
                pltpu.sync_copy(lab_hbm.at[pl.ds(tbase, tail)],
                                tlab_v.at[pl.ds(0, tail)])

                @plsc.parallel_loop(0, ntail * _LANES, step=_LANES)
                def _(s):
                    chunk(tconf_v, tpred_v, tlab_v, s)

        pltpu.sync_copy(hist_v, out_hbm.at[pl.ds(wid * hsz, hsz)])

    return k(conf, pred, lab)


def _ece_body(p_ref, o_ref, *, n_total):
    x = p_ref[...]
    rr = jax.lax.broadcasted_iota(jnp.int32, (128, 128), 0) % 4
    cc = jax.lax.broadcasted_iota(jnp.int32, (128, 128), 1)
    r = rr * 8 + cc // 16
    ispk = r < N_BINS
    isconf = (r >= N_BINS) & (r < 2 * N_BINS)
    fl = jnp.floor(x)
    cnt_e = jnp.where(ispk, fl, 0.0)
    acc_e = jnp.where(ispk, (x - fl) * 4096.0, 0.0)
    conf_e = jnp.where(isconf, x, 0.0)

    def redu(z):
        z1 = jnp.sum(z.reshape(32, 4, 128), axis=0)
        z2 = jnp.sum(z1.reshape(4, 8, 16), axis=-1)
        return z2.reshape(32, 1)

    cnt = redu(cnt_e)[0:N_BINS]
    sacc = redu(acc_e)[0:N_BINS]
    sconf = redu(conf_e)[N_BINS:2 * N_BINS]
    safe = jnp.maximum(cnt, 1.0)
    prop = cnt / n_total
    contrib = jnp.abs(sconf / safe - sacc / safe) * prop
    ece = jnp.sum(jnp.where(prop > 0.0, contrib, 0.0))
    o_ref[0] = ece


def kernel(confidences, predictions, labels):
    n = confidences.shape[0]
    conf = confidences.astype(jnp.float32)
    pred = predictions.astype(jnp.int32)
    lab = labels.astype(jnp.int32)
    if n % _LANES:
        npad = -(-n // _LANES) * _LANES
        conf = jnp.pad(conf, (0, npad - n))
        pred = jnp.pad(pred, (0, npad - n))
        lab = jnp.pad(lab, (0, npad - n))
        m = npad
    else:
        m = n
    ch = (m // (_NW * _LANES)) * _LANES
    tail = m - _NW * ch

    flat = _sc_hist(conf, pred, lab, ch, tail)
    parts = flat.reshape(128, 128)

    ece = pl.pallas_call(
        functools.partial(_ece_body, n_total=float(n)),
        out_shape=jax.ShapeDtypeStruct((1,), jnp.float32),
        out_specs=pl.BlockSpec(memory_space=pltpu.SMEM),
    )(parts)
    return ece
